# baseline (speedup 1.0000x reference)
import functools

import jax
import jax.numpy as jnp
from jax import lax
from jax.experimental import pallas as pl
from jax.experimental.pallas import tpu as pltpu
from jax.experimental.pallas import tpu_sc as plsc

VOCAB = 1000000
DIM = 32
FIELDS = 26
BATCH = 4096

_INFO = plsc.get_sparse_core_info()
_NC, _NS = _INFO.num_cores, _INFO.num_subcores
_NW = _NC * _NS
_BW = BATCH // _NW
_TR = FIELDS * DIM // 8
_VBLK = 65536
_GRP = _VBLK // 4
_TGRID = -(-VOCAB // _VBLK)
_VPAD = _TGRID * _VBLK
_SB = _VBLK.bit_length() - 1
_SG = _GRP.bit_length() - 1


def _transpose_body(x_ref, o_ref):
    x = x_ref[...]
    stacked = jnp.concatenate(
        [x[:, g * _GRP:(g + 1) * _GRP] for g in range(4)], axis=0
    )
    o_ref[...] = stacked.T


_transpose_call = pl.pallas_call(
    _transpose_body,
    grid=(_TGRID,),
    in_specs=[pl.BlockSpec((DIM, _VBLK), lambda g: (0, g))],
    out_specs=pl.BlockSpec((_VBLK * DIM // 128, 128), lambda g: (g, 0)),
    out_shape=jax.ShapeDtypeStruct((_VPAD * DIM // 128, 128), jnp.float32),
)


@functools.partial(
    pl.kernel,
    mesh=plsc.VectorSubcoreMesh(core_axis_name="c", subcore_axis_name="s"),
    out_type=jax.ShapeDtypeStruct((_TR, _NW, 8, _BW), jnp.float32),
    compiler_params=pltpu.CompilerParams(
        use_tc_tiling_on_sc=False, needs_layout_passes=False
    ),
    scratch_types=[
        pltpu.VMEM((FIELDS, _BW), jnp.int32),
        pltpu.VMEM((_BW, DIM), jnp.float32),
        pltpu.VMEM((_BW, DIM), jnp.float32),
        pltpu.VMEM((4, 8, _BW), jnp.float32),
        pltpu.VMEM((4, 8, _BW), jnp.float32),
        pltpu.SemaphoreType.DMA,
        pltpu.SemaphoreType.DMA,
        pltpu.SemaphoreType.DMA,
        pltpu.SemaphoreType.DMA,
    ],
)
def _gather_kernel(
    idx_hbm, table_hbm, o4, idx_v, rfa, rfb, tla, tlb, gsa, gsb, wsa, wsb
):
    wid = lax.axis_index("s") * _NC + lax.axis_index("c")
    pltpu.sync_copy(idx_hbm.at[wid], idx_v)

    def gather(f, rf, sem):
        return pltpu.async_copy(table_hbm.at[idx_v.at[f]], rf, sem)

    def gather_wait(f, rf, sem):
        pltpu.make_async_copy(table_hbm.at[idx_v.at[f]], rf, sem).wait()

    def write(f, tl, sem):
        return pltpu.make_async_copy(tl, o4.at[pl.ds(4 * f, 4), wid], sem)

    def shuffle(rf, tl):
        for t4 in range(4):
            for cs in range(8):
                col = jnp.full((16,), 8 * t4 + cs, jnp.int32)
                for h in range(8):
                    row = lax.iota(jnp.int32, 16) + 16 * h
                    tl[t4, cs, pl.ds(16 * h, 16)] = plsc.load_gather(
                        rf, [row, col]
                    )

    gather(0, rfa, gsa)

    def body(k, carry):
        fa = 2 * k
        gather(fa + 1, rfb, gsb)
        gather_wait(fa, rfa, gsa)

        @pl.when(k > 0)
        def _():
            write(fa - 2, tla, wsa).wait()

        shuffle(rfa, tla)
        wa = write(fa, tla, wsa)
        wa.start()

        @pl.when(k < FIELDS // 2 - 1)
        def _():
            gather(fa + 2, rfa, gsa)

        gather_wait(fa + 1, rfb, gsb)

        @pl.when(k > 0)
        def _():
            write(fa - 1, tlb, wsb).wait()

        shuffle(rfb, tlb)
        wb = write(fa + 1, tlb, wsb)
        wb.start()
        return carry

    lax.fori_loop(0, FIELDS // 2, body, 0)
    write(FIELDS - 2, tla, wsa).wait()
    write(FIELDS - 1, tlb, wsb).wait()


def kernel(indices, embedding_table):
    u = (
        ((indices >> _SB) << _SB)
        | ((indices & (_GRP - 1)) << 2)
        | ((indices >> _SG) & 3)
    )
    idx3 = u.reshape(FIELDS, _NW, _BW).swapaxes(0, 1)
    t = _transpose_call(embedding_table.T)
    t = t.reshape(_VPAD, DIM)
    o4 = _gather_kernel(idx3, t)
    return o4.transpose(0, 2, 1, 3).reshape(FIELDS * DIM, BATCH).T


# speedup vs baseline: 1.3435x; 1.3435x over previous
import functools

import jax
import jax.numpy as jnp
from jax import lax
from jax.experimental import pallas as pl
from jax.experimental.pallas import tpu as pltpu
from jax.experimental.pallas import tpu_sc as plsc

VOCAB = 1000000
DIM = 32
FIELDS = 26
BATCH = 4096

_INFO = plsc.get_sparse_core_info()
_NC, _NS = _INFO.num_cores, _INFO.num_subcores
_NW = _NC * _NS
_TOTAL = FIELDS * BATCH
_PER_W = _TOTAL // _NW
_CHUNK = 128
_NCHUNK = _PER_W // _CHUNK


@functools.partial(
    pl.kernel,
    mesh=plsc.VectorSubcoreMesh(core_axis_name="c", subcore_axis_name="s"),
    out_type=jax.ShapeDtypeStruct((_TOTAL, DIM), jnp.float32),
    compiler_params=pltpu.CompilerParams(use_tc_tiling_on_sc=False),
    scratch_types=[
        pltpu.VMEM((_NCHUNK, _CHUNK), jnp.int32),
        pltpu.VMEM((_PER_W, DIM), jnp.float32),
        pltpu.SemaphoreType.DMA,
        pltpu.SemaphoreType.DMA,
    ],
)
def _gather_kernel(idx_hbm, table_hbm, out_hbm, idx_v, rows_v, gsem, wsem):
    wid = lax.axis_index("s") * _NC + lax.axis_index("c")
    pltpu.sync_copy(idx_hbm.at[wid], idx_v)
    gathers, writes = [], []
    for j in range(_NCHUNK):
        gathers.append(
            pltpu.async_copy(
                table_hbm.at[idx_v.at[j]],
                rows_v.at[pl.ds(j * _CHUNK, _CHUNK)],
                gsem,
            )
        )
    for j in range(_NCHUNK):
        gathers[j].wait()
        writes.append(
            pltpu.async_copy(
                rows_v.at[pl.ds(j * _CHUNK, _CHUNK)],
                out_hbm.at[pl.ds(wid * _PER_W + j * _CHUNK, _CHUNK)],
                wsem,
            )
        )
    for cp in writes:
        cp.wait()


_VBLK = 65536
_GRP = _VBLK // 4
_TGRID = -(-VOCAB // _VBLK)
_VPAD = _TGRID * _VBLK
_SB = _VBLK.bit_length() - 1
_SG = _GRP.bit_length() - 1


def _transpose_body(x_ref, o_ref):
    x = x_ref[...]
    stacked = jnp.concatenate(
        [x[:, g * _GRP:(g + 1) * _GRP] for g in range(4)], axis=0
    )
    o_ref[...] = stacked.T


_transpose_call = pl.pallas_call(
    _transpose_body,
    grid=(_TGRID,),
    in_specs=[pl.BlockSpec((DIM, _VBLK), lambda g: (0, g))],
    out_specs=pl.BlockSpec((_VBLK * DIM // 128, 128), lambda g: (g, 0)),
    out_shape=jax.ShapeDtypeStruct((_VPAD * DIM // 128, 128), jnp.float32),
)


def kernel(indices, embedding_table):
    vt = indices.T
    u = ((vt >> _SB) << _SB) | ((vt & (_GRP - 1)) << 2) | ((vt >> _SG) & 3)
    idx2 = u.reshape(_NW, _NCHUNK, _CHUNK)
    t = _transpose_call(embedding_table.T)
    t = t.reshape(_VPAD, DIM)
    out = _gather_kernel(idx2, t)
    return out.reshape(BATCH, FIELDS * DIM)
